# baseline (speedup 1.0000x reference)
import jax
import jax.numpy as jnp
from jax.experimental import pallas as pl
from jax.experimental.pallas import tpu as pltpu

_B = 512
_BBLK = 128
_NBLK = _B // _BBLK
_N = 68
_NP = 72
_K = 15
_D = 128
_F = _K * _D
_T = _K * 3
_E = _F + _T
_INV_T2 = 1.0 / (200.0 * 200.0)


def _cos_classifier_body(emb_hbm, pw_hbm, out_ref, emb_buf, pw_buf,
                         sems, csem):
    _H = _BBLK // 2
    def _emb_half(i, h):
        return pltpu.make_async_copy(
            emb_hbm.at[pl.ds(i * _BBLK + h * _H, _H), :],
            emb_buf.at[i, pl.ds(h * _H, _H), :], sems.at[i, h])

    pw_cp = pltpu.make_async_copy(pw_hbm, pw_buf, csem)
    pw_cp.start()
    for h in range(2):
        _emb_half(0, h).start()
    pw_cp.wait()
    p = pw_buf[:, :_F]
    ones_f = jnp.ones((1, _F), dtype=jnp.float32)
    pn2 = jax.lax.dot_general(p * p, ones_f, (((1,), (1,)), ((), ())),
                              preferred_element_type=jnp.float32)
    rp = jax.lax.rsqrt(jnp.maximum(pn2, 1e-24))

    ptail = jnp.pad(pw_buf[:, _F:], ((0, _NP - _N), (0, 0)))
    ik = jax.lax.broadcasted_iota(jnp.int32, (_K, 1, _T), 0)
    ic = jax.lax.broadcasted_iota(jnp.int32, (_K, 1, _T), 2)
    bmask = jnp.where(ik == ic // 3, -2.0 * _INV_T2, 0.0)
    A = (jnp.broadcast_to(ptail[None], (_K, _NP, _T)) * bmask
         ).reshape(_K * _NP, _T)
    ones_t = jnp.ones((1, _T), dtype=jnp.float32)
    pn2a = jax.lax.dot_general(A * A, ones_t, (((1,), (1,)), ((), ())),
                               preferred_element_type=jnp.float32)
    pn2a = (pn2a * (0.25 / _INV_T2)).reshape(_K, _NP, 1)
    io = jax.lax.broadcasted_iota(jnp.int32, (_K, _T), 0)
    jo = jax.lax.broadcasted_iota(jnp.int32, (_K, _T), 1)
    O = jnp.where(io == jo // 3, _INV_T2, 0.0)

    for i in range(_NBLK):
        for h in range(2):
            _emb_half(i, h).wait()
        if i + 1 < _NBLK:
            for h in range(2):
                _emb_half(i + 1, h).start()
        x = emb_buf[i, :, :_F]
        xn2 = jax.lax.dot_general(ones_f, x * x, (((1,), (1,)), ((), ())),
                                  preferred_element_type=jnp.float32)
        rx = jax.lax.rsqrt(jnp.maximum(xn2, 1e-24))

        et = emb_buf[i, :, _F:]
        xn2a = jax.lax.dot_general(O, et * et, (((1,), (1,)), ((), ())),
                                   preferred_element_type=jnp.float32)
        cross = jax.lax.dot_general(A, et, (((1,), (1,)), ((), ())),
                                    preferred_element_type=jnp.float32)
        d2 = (xn2a.reshape(_K, 1, _BBLK) + cross.reshape(_K, _NP, _BBLK)
              + pn2a)
        t = jnp.sqrt(jnp.abs(d2))
        m = jnp.max(t, axis=0, keepdims=True)
        e = jnp.exp(t - m)
        s = jnp.sum(e, axis=0, keepdims=True)
        w2 = e * ((240.0 * rx[None]) / s)

        acc = jnp.zeros((_N, _BBLK), dtype=jnp.float32)
        for k in range(_K):
            sk = jax.lax.dot_general(
                p[:, k * _D:(k + 1) * _D], x[:, k * _D:(k + 1) * _D],
                dimension_numbers=(((1,), (1,)), ((), ())),
                preferred_element_type=jnp.float32,
            )
            acc = acc + w2[k, :_N, :] * sk
        out_ref[pl.ds(i * _BBLK, _BBLK), :] = jnp.transpose(acc * rp)


@jax.jit
def kernel(emb, proto_w):
    return pl.pallas_call(
        _cos_classifier_body,
        in_specs=[
            pl.BlockSpec(memory_space=pltpu.MemorySpace.HBM),
            pl.BlockSpec(memory_space=pltpu.MemorySpace.HBM),
        ],
        scratch_shapes=[
            pltpu.VMEM((_NBLK, _BBLK, _E), jnp.float32),
            pltpu.VMEM((_N, _E), jnp.float32),
            pltpu.SemaphoreType.DMA((_NBLK, 2)),
            pltpu.SemaphoreType.DMA,
        ],
        out_shape=jax.ShapeDtypeStruct((_B, _N), jnp.float32),
    )(emb, proto_w)


# speedup vs baseline: 1.0738x; 1.0738x over previous
import jax
import jax.numpy as jnp
from jax.experimental import pallas as pl
from jax.experimental.pallas import tpu as pltpu

_B = 512
_BBLK = 128
_NBLK = _B // _BBLK
_N = 68
_NP = 72
_K = 15
_D = 128
_F = _K * _D
_T = _K * 3
_E = _F + _T
_INV_T2 = 1.0 / (200.0 * 200.0)


def _cos_classifier_body(emb_hbm, pw_hbm, out_ref, emb_buf, pw_buf,
                         sems, sem0h, csem):
    _H = _BBLK // 2
    def _e0_half(h):
        return pltpu.make_async_copy(
            emb_hbm.at[pl.ds(h * _H, _H), :],
            emb_buf.at[0, pl.ds(h * _H, _H), :], sem0h.at[h])

    pw_cp = pltpu.make_async_copy(pw_hbm, pw_buf, csem)
    pw_cp.start()
    _e0_half(0).start()
    _e0_half(1).start()
    pltpu.make_async_copy(
        emb_hbm.at[pl.ds(_BBLK, _BBLK), :], emb_buf.at[1],
        sems.at[1]).start()

    pw_cp.wait()
    p = pw_buf[:, :_F]
    ones_f = jnp.ones((1, _F), dtype=jnp.float32)
    pn2 = jax.lax.dot_general(p * p, ones_f, (((1,), (1,)), ((), ())),
                              preferred_element_type=jnp.float32)
    rp = jax.lax.rsqrt(jnp.maximum(pn2, 1e-24))

    ptail = jnp.pad(pw_buf[:, _F:], ((0, _NP - _N), (0, 0)))
    ik = jax.lax.broadcasted_iota(jnp.int32, (_K, 1, _T), 0)
    ic = jax.lax.broadcasted_iota(jnp.int32, (_K, 1, _T), 2)
    bmask = jnp.where(ik == ic // 3, -2.0 * _INV_T2, 0.0)
    A = (jnp.broadcast_to(ptail[None], (_K, _NP, _T)) * bmask
         ).reshape(_K * _NP, _T)
    ones_t = jnp.ones((1, _T), dtype=jnp.float32)
    pn2a = jax.lax.dot_general(A * A, ones_t, (((1,), (1,)), ((), ())),
                               preferred_element_type=jnp.float32)
    pn2a = (pn2a * (0.25 / _INV_T2)).reshape(_K, _NP, 1)
    io = jax.lax.broadcasted_iota(jnp.int32, (_K, _T), 0)
    jo = jax.lax.broadcasted_iota(jnp.int32, (_K, _T), 1)
    O = jnp.where(io == jo // 3, _INV_T2, 0.0)

    for i in range(_NBLK):
        if i == 0:
            _e0_half(0).wait()
            _e0_half(1).wait()
        else:
            pltpu.make_async_copy(
                emb_hbm.at[pl.ds(i * _BBLK, _BBLK), :], emb_buf.at[i],
                sems.at[i]).wait()
        if i + 2 < _NBLK:
            pltpu.make_async_copy(
                emb_hbm.at[pl.ds((i + 2) * _BBLK, _BBLK), :],
                emb_buf.at[i + 2], sems.at[i + 2]).start()
        x = emb_buf[i, :, :_F]
        xn2 = jax.lax.dot_general(ones_f, x * x, (((1,), (1,)), ((), ())),
                                  preferred_element_type=jnp.float32)
        rx = jax.lax.rsqrt(jnp.maximum(xn2, 1e-24))

        et = emb_buf[i, :, _F:]
        xn2a = jax.lax.dot_general(O, et * et, (((1,), (1,)), ((), ())),
                                   preferred_element_type=jnp.float32)
        cross = jax.lax.dot_general(A, et, (((1,), (1,)), ((), ())),
                                    preferred_element_type=jnp.float32)
        d2 = (xn2a.reshape(_K, 1, _BBLK) + cross.reshape(_K, _NP, _BBLK)
              + pn2a)
        t = jnp.sqrt(jnp.abs(d2))
        m = jnp.max(t, axis=0, keepdims=True)
        e = jnp.exp(t - m)
        s = jnp.sum(e, axis=0, keepdims=True)
        w2 = e * ((240.0 * rx[None]) / s)

        acc = jnp.zeros((_N, _BBLK), dtype=jnp.float32)
        for k in range(_K):
            sk = jax.lax.dot_general(
                p[:, k * _D:(k + 1) * _D], x[:, k * _D:(k + 1) * _D],
                dimension_numbers=(((1,), (1,)), ((), ())),
                preferred_element_type=jnp.float32,
            )
            acc = acc + w2[k, :_N, :] * sk
        out_ref[pl.ds(i * _BBLK, _BBLK), :] = jnp.transpose(acc * rp)


@jax.jit
def kernel(emb, proto_w):
    return pl.pallas_call(
        _cos_classifier_body,
        in_specs=[
            pl.BlockSpec(memory_space=pltpu.MemorySpace.HBM),
            pl.BlockSpec(memory_space=pltpu.MemorySpace.HBM),
        ],
        scratch_shapes=[
            pltpu.VMEM((_NBLK, _BBLK, _E), jnp.float32),
            pltpu.VMEM((_N, _E), jnp.float32),
            pltpu.SemaphoreType.DMA((_NBLK,)),
            pltpu.SemaphoreType.DMA((2,)),
            pltpu.SemaphoreType.DMA,
        ],
        out_shape=jax.ShapeDtypeStruct((_B, _N), jnp.float32),
    )(emb, proto_w)
